# baseline (speedup 1.0000x reference)
import jax
import jax.numpy as jnp
from jax.experimental import pallas as pl
from jax.experimental.pallas import tpu as pltpu

_S = 60000
_SBLK = 8192
_NSTEPS = (_S + _SBLK - 1) // _SBLK
_TAILW = 2688


def _silu(h):
    return h * jax.nn.sigmoid(h)


def _bn(h, g, b):
    m = jnp.mean(h, axis=0, keepdims=True)
    v = jnp.mean((h - m) ** 2, axis=0, keepdims=True)
    return g * (h - m) * jax.lax.rsqrt(v + 1e-5) + b


def _dot_t(a, b):
    return jax.lax.dot_general(
        a, b, (((1,), (1,)), ((), ())), preferred_element_type=jnp.float32)


def _mlp_kernel(x_ref, w1_ref, b1_ref, g1_ref, bt1_ref, w2_ref, b2_ref,
                w3_ref, b3_ref, g2_ref, bt2_ref, w4_ref, b4_ref,
                g3_ref, bt3_ref, w5_ref, b5_ref, out_ref, acc_ref):
    i = pl.program_id(0)

    @pl.when(i == 0)
    def _init():
        acc_ref[...] = jnp.zeros_like(acc_ref)

    @pl.when(i < _NSTEPS - 1)
    def _body():
        acc_ref[...] += _dot_t(x_ref[...].astype(jnp.bfloat16),
                               w1_ref[...].astype(jnp.bfloat16))

    @pl.when(i == _NSTEPS - 1)
    def _tail():
        col = jax.lax.broadcasted_iota(jnp.int32, (1, _TAILW), 1)
        valid = col < (_S - i * _SBLK)
        xb = jnp.where(valid, x_ref[:, :_TAILW], 0.0).astype(jnp.bfloat16)
        wb = jnp.where(valid, w1_ref[:, :_TAILW], 0.0).astype(jnp.bfloat16)

        vec = lambda r: r[...].reshape(1, -1)
        h = acc_ref[...] + _dot_t(xb, wb) + vec(b1_ref)
        h = _silu(_bn(h, vec(g1_ref), vec(bt1_ref)))
        h = _silu(_dot_t(h, w2_ref[...]) + vec(b2_ref))
        h = _dot_t(h, w3_ref[...]) + vec(b3_ref)
        h = _silu(_bn(h, vec(g2_ref), vec(bt2_ref)))
        h = _dot_t(h, w4_ref[...]) + vec(b4_ref)
        h = _silu(_bn(h, vec(g3_ref), vec(bt3_ref)))
        out_ref[...] = _dot_t(h, w5_ref[...]) + vec(b5_ref)


def kernel(x, W1, b1, g1, bt1, W2, b2, W3, b3, g2, bt2, W4, b4, g3, bt3,
           W5, b5):
    B, S = x.shape
    D1, D2, D3 = W2.shape[0], W3.shape[0], W4.shape[0]

    once = pl.Buffered(buffer_count=1)
    fullv = lambda d: pl.BlockSpec((d,), lambda i: (0,), pipeline_mode=once)
    fullm = lambda r, c: pl.BlockSpec((r, c), lambda i: (0, 0),
                                      pipeline_mode=once)
    in_specs = [
        pl.BlockSpec((B, _SBLK), lambda i: (0, i)),
        pl.BlockSpec((D1, _SBLK), lambda i: (0, i)),
        fullv(D1), fullv(D1), fullv(D1),
        fullm(D1, D1), fullv(D1),
        fullm(D2, D1), fullv(D2),
        fullv(D2), fullv(D2),
        fullm(D3, D2), fullv(D3),
        fullv(D3), fullv(D3),
        fullm(D3, D3), fullv(D3),
    ]
    out = pl.pallas_call(
        _mlp_kernel,
        grid=(_NSTEPS,),
        in_specs=in_specs,
        out_specs=pl.BlockSpec((B, D3), lambda i: (0, 0)),
        out_shape=jax.ShapeDtypeStruct((B, D3), jnp.float32),
        scratch_shapes=[pltpu.VMEM((B, D1), jnp.float32)],
    )(x, W1, b1, g1, bt1, W2, b2, W3, b3, g2, bt2, W4, b4, g3, bt3, W5, b5)
    return out
